# baseline (speedup 1.0000x reference)
import functools

import jax
import jax.numpy as jnp
from jax import lax
from jax.experimental import pallas as pl
from jax.experimental.pallas import tpu as pltpu
from jax.experimental.pallas import tpu_sc as plsc

N_ATOMS = 100000
FEAT = 128
NC, NS = 2, 16
NW = NC * NS
ROWS_W = N_ATOMS // NW
CHUNK = 125
CHUNK_PAD = 128
NCHUNK = ROWS_W // CHUNK
NBUF = 5

_mesh = plsc.VectorSubcoreMesh(core_axis_name="c", subcore_axis_name="s")


@functools.partial(
    pl.kernel,
    out_type=jax.ShapeDtypeStruct((N_ATOMS, FEAT), jnp.float32),
    mesh=_mesh,
    scratch_types=[
        pltpu.VMEM((NCHUNK, CHUNK_PAD), jnp.int32),
        [pltpu.VMEM((CHUNK_PAD, FEAT), jnp.float32) for _ in range(NBUF)],
        [pltpu.SemaphoreType.DMA for _ in range(NBUF)],
        [pltpu.SemaphoreType.DMA for _ in range(NBUF)],
    ],
    compiler_params=pltpu.CompilerParams(use_tc_tiling_on_sc=False),
)
def _embed(idx_hbm, table_hbm, out_hbm, idx_v, bufs, gsems, ssems):
    wid = lax.axis_index("s") * NC + lax.axis_index("c")
    base = wid * ROWS_W
    pltpu.sync_copy(idx_hbm.at[wid], idx_v)

    def outer(k, carry):
        for b in range(NBUF):
            j = k * NBUF + b

            @pl.when(k > 0)
            def _():
                pltpu.make_async_copy(
                    bufs[b].at[pl.ds(0, CHUNK)],
                    out_hbm.at[pl.ds(base, CHUNK)],
                    ssems[b],
                ).wait()

            pltpu.make_async_copy(
                table_hbm.at[idx_v.at[j]], bufs[b], gsems[b]
            ).start()
        for b in range(NBUF):
            j = k * NBUF + b
            pltpu.make_async_copy(
                table_hbm.at[idx_v.at[j]], bufs[b], gsems[b]
            ).wait()
            pltpu.make_async_copy(
                bufs[b].at[pl.ds(0, CHUNK)],
                out_hbm.at[pl.ds(base + j * CHUNK, CHUNK)],
                ssems[b],
            ).start()
        return carry

    lax.fori_loop(0, NCHUNK // NBUF, outer, 0)
    for b in range(NBUF):
        pltpu.make_async_copy(
            bufs[b].at[pl.ds(0, CHUNK)],
            out_hbm.at[pl.ds(base, CHUNK)],
            ssems[b],
        ).wait()


def kernel(atomic_numbers, table):
    idx = atomic_numbers.reshape(NW, NCHUNK, CHUNK).astype(jnp.int32)
    idx = jnp.pad(idx, ((0, 0), (0, 0), (0, CHUNK_PAD - CHUNK)))
    out = _embed(idx, table)
    return out.reshape(N_ATOMS, 1, FEAT)


# speedup vs baseline: 5.4453x; 5.4453x over previous
import functools

import jax
import jax.numpy as jnp
from jax import lax
from jax.experimental import pallas as pl
from jax.experimental.pallas import tpu as pltpu
from jax.experimental.pallas import tpu_sc as plsc

N_ATOMS = 100000
FEAT = 128
NC, NS = 2, 16
NW = NC * NS
CHUNK = 128
ROWS_W = 3128
ROWS_LAST = N_ATOMS - (NW - 1) * ROWS_W
NCHUNK = 25
NBUF = 5

_mesh = plsc.VectorSubcoreMesh(core_axis_name="c", subcore_axis_name="s")


@functools.partial(
    pl.kernel,
    out_type=jax.ShapeDtypeStruct((N_ATOMS, FEAT), jnp.float32),
    mesh=_mesh,
    scratch_types=[
        pltpu.VMEM((NCHUNK, CHUNK), jnp.int32),
        pltpu.VMEM_SHARED((94, FEAT), jnp.float32),
        [pltpu.VMEM((CHUNK, FEAT), jnp.float32) for _ in range(NBUF)],
        [pltpu.SemaphoreType.DMA for _ in range(NBUF)],
        [pltpu.SemaphoreType.DMA for _ in range(NBUF)],
        pltpu.SemaphoreType.DMA,
    ],
    compiler_params=pltpu.CompilerParams(use_tc_tiling_on_sc=False),
)
def _embed(idx_hbm, table_hbm, out_hbm, idx_v, table_v, bufs, gsems, ssems, isem):
    wid = lax.axis_index("s") * NC + lax.axis_index("c")
    base = wid * ROWS_W
    rows = jnp.where(wid == NW - 1, ROWS_LAST, ROWS_W)

    def chunk_off(j):
        return jnp.minimum(j * CHUNK, rows - CHUNK)

    def stage(j, carry):
        pltpu.make_async_copy(
            idx_hbm.at[pl.ds(base + chunk_off(j), CHUNK)],
            idx_v.at[j],
            isem,
        ).start()
        return carry

    lax.fori_loop(0, NCHUNK, stage, 0)

    @pl.when(lax.axis_index("s") == 0)
    def _():
        pltpu.sync_copy(table_hbm, table_v)

    plsc.subcore_barrier()

    def drain(j, carry):
        pltpu.make_async_copy(
            idx_hbm.at[pl.ds(base, CHUNK)], idx_v.at[0], isem
        ).wait()
        return carry

    lax.fori_loop(0, NCHUNK, drain, 0)

    def outer(k, carry):
        for b in range(NBUF):
            j = k * NBUF + b

            @pl.when(k > 0)
            def _():
                pltpu.make_async_copy(
                    bufs[b], out_hbm.at[pl.ds(base, CHUNK)], ssems[b]
                ).wait()

            pltpu.make_async_copy(
                table_v.at[idx_v.at[j]], bufs[b], gsems[b]
            ).start()
        for b in range(NBUF):
            j = k * NBUF + b
            pltpu.make_async_copy(
                table_v.at[idx_v.at[j]], bufs[b], gsems[b]
            ).wait()
            pltpu.make_async_copy(
                bufs[b],
                out_hbm.at[pl.ds(base + chunk_off(j), CHUNK)],
                ssems[b],
            ).start()
        return carry

    lax.fori_loop(0, NCHUNK // NBUF, outer, 0)
    for b in range(NBUF):
        pltpu.make_async_copy(
            bufs[b], out_hbm.at[pl.ds(base, CHUNK)], ssems[b]
        ).wait()


def kernel(atomic_numbers, table):
    out = _embed(atomic_numbers.astype(jnp.int32).reshape(N_ATOMS), table)
    return out.reshape(N_ATOMS, 1, FEAT)


# speedup vs baseline: 5.4967x; 1.0094x over previous
import functools

import jax
import jax.numpy as jnp
from jax import lax
from jax.experimental import pallas as pl
from jax.experimental.pallas import tpu as pltpu
from jax.experimental.pallas import tpu_sc as plsc

N_ATOMS = 100000
FEAT = 128
NC, NS = 2, 16
NW = NC * NS
CHUNK = 128
ROWS_W = 3128
ROWS_LAST = N_ATOMS - (NW - 1) * ROWS_W
NCHUNK = 25
NBUF = 5

_mesh = plsc.VectorSubcoreMesh(core_axis_name="c", subcore_axis_name="s")


@functools.partial(
    pl.kernel,
    out_type=jax.ShapeDtypeStruct((N_ATOMS, FEAT), jnp.float32),
    mesh=_mesh,
    scratch_types=[
        pltpu.VMEM((NCHUNK * CHUNK,), jnp.int32),
        pltpu.VMEM_SHARED((94, FEAT), jnp.float32),
        [pltpu.VMEM((CHUNK, FEAT), jnp.float32) for _ in range(NBUF)],
        [pltpu.SemaphoreType.DMA for _ in range(NBUF)],
        [pltpu.SemaphoreType.DMA for _ in range(NBUF)],
        pltpu.SemaphoreType.DMA,
    ],
    compiler_params=pltpu.CompilerParams(use_tc_tiling_on_sc=False),
)
def _embed(idx_hbm, table_hbm, out_hbm, idx_v, table_v, bufs, gsems, ssems, isem):
    wid = lax.axis_index("s") * NC + lax.axis_index("c")
    base = pl.multiple_of(wid * ROWS_W, 8)
    rows = jnp.where(wid == NW - 1, ROWS_LAST, ROWS_W)

    def chunk_off(j):
        return pl.multiple_of(jnp.minimum(j * CHUNK, rows - CHUNK), 8)

    STAGE_HEAD = 2904
    STAGE_TAIL = 224
    tail_off = pl.multiple_of(rows - STAGE_TAIL, 8)
    cp_head = pltpu.make_async_copy(
        idx_hbm.at[pl.ds(base, STAGE_HEAD)], idx_v.at[pl.ds(0, STAGE_HEAD)], isem
    )
    cp_tail = pltpu.make_async_copy(
        idx_hbm.at[pl.ds(base + tail_off, STAGE_TAIL)],
        idx_v.at[pl.ds(tail_off, STAGE_TAIL)],
        isem,
    )
    cp_head.start()
    cp_tail.start()

    @pl.when(lax.axis_index("s") == 0)
    def _():
        pltpu.sync_copy(table_hbm, table_v)

    plsc.subcore_barrier()
    cp_head.wait()
    cp_tail.wait()

    def outer(k, carry):
        for b in range(NBUF):
            j = k * NBUF + b

            @pl.when(k > 0)
            def _():
                pltpu.make_async_copy(
                    bufs[b], out_hbm.at[pl.ds(base, CHUNK)], ssems[b]
                ).wait()

            pltpu.make_async_copy(
                table_v.at[idx_v.at[pl.ds(chunk_off(j), CHUNK)]], bufs[b], gsems[b]
            ).start()
        for b in range(NBUF):
            j = k * NBUF + b
            pltpu.make_async_copy(
                table_v.at[idx_v.at[pl.ds(chunk_off(j), CHUNK)]], bufs[b], gsems[b]
            ).wait()
            pltpu.make_async_copy(
                bufs[b],
                out_hbm.at[pl.ds(base + chunk_off(j), CHUNK)],
                ssems[b],
            ).start()
        return carry

    lax.fori_loop(0, NCHUNK // NBUF, outer, 0)
    for b in range(NBUF):
        pltpu.make_async_copy(
            bufs[b], out_hbm.at[pl.ds(base, CHUNK)], ssems[b]
        ).wait()


def kernel(atomic_numbers, table):
    out = _embed(atomic_numbers.astype(jnp.int32).reshape(N_ATOMS), table)
    return out.reshape(N_ATOMS, 1, FEAT)
